# baseline (speedup 1.0000x reference)
import jax
import jax.numpy as jnp
from jax.experimental import pallas as pl

_SPLITS = (0, 6400, 8000, 8400, 8500, 8525)
_SCALES = (0.1, 0.2, 0.375, 0.55, 0.725)
_N_LEVELS = 5
_K = 9
_N_OBJ = 8
_N_CLASSES = 80
_BIG_F = 1e30

_L1P = (0.003662242172958829, -0.022628007027180385, 0.06573552525611641,
        -0.12447194531797226, 0.1842138633882333, -0.24618967713387793,
        0.3327853379900572, -0.4999589446838273, 0.9999987830866915,
        6.057848218605543e-09)


def _iota(shape, dim):
    return jax.lax.broadcasted_iota(jnp.int32, shape, dim)


def _lane_to_sublane(v, n):
    r = _iota((n, n), 0)
    c = _iota((n, n), 1)
    vb = jnp.broadcast_to(v, (n, n))
    zero = jnp.zeros((), v.dtype)
    return jnp.sum(jnp.where(r == c, vb, zero), axis=1, keepdims=True)


def _log1p01(e):
    r = jnp.full_like(e, _L1P[0])
    for c in _L1P[1:]:
        r = r * e + c
    return r


def _focal_terms(x):
    e = jnp.exp(-jnp.abs(x))
    l1pe = _log1p01(e)
    rc = 1.0 / (1.0 + e)
    p = jnp.where(x >= 0.0, rc, e * rc)
    sp_pos = jnp.maximum(x, 0.0) + l1pe
    sp_neg = jnp.maximum(-x, 0.0) + l1pe
    return p, sp_pos, sp_neg


def _atan_nonneg(x):
    inv = x > 1.0
    y = jnp.where(inv, 1.0 / jnp.where(inv, x, 1.0), x)
    y = y / (1.0 + jnp.sqrt(1.0 + y * y))
    y = y / (1.0 + jnp.sqrt(1.0 + y * y))
    t = y * y
    s = y * (1.0 + t * (-1.0 / 3.0 + t * (1.0 / 5.0
                                          + t * (-1.0 / 7.0 + t / 9.0))))
    a = 4.0 * s
    return jnp.where(inv, jnp.pi / 2.0 - a, a)


def _body(scores_ref, pos_ref, locs_ref, boxes_ref, labels_ref, priors_ref,
          out_ref):
    nb = scores_ref.shape[0]
    n45 = _N_LEVELS * _K
    nlane = nb * n45

    boxes = boxes_ref[...]
    bx1 = boxes[:, 0:1]
    by1 = boxes[:, 1:2]
    bx2 = boxes[:, 2:3]
    by2 = boxes[:, 3:4]
    gcx = (bx1 + bx2) * 0.5
    gcy = (by1 + by2) * 0.5
    area_a = (bx2 - bx1) * (by2 - by1)

    px = priors_ref[0:1, :]
    py = priors_ref[1:2, :]
    pw = priors_ref[2:3, :]
    ph = priors_ref[3:4, :]

    nrows = nb * _N_OBJ

    lvl = []
    for l in range(_N_LEVELS):
        s0, s1 = _SPLITS[l], _SPLITS[l + 1]
        npl = s1 - s0
        pxl = px[:, s0:s1]
        pyl = py[:, s0:s1]
        pwl = pw[:, s0:s1]
        phl = ph[:, s0:s1]
        plx1 = pxl - pwl / 2.0
        ply1 = pyl - phl / 2.0
        plx2 = pxl + pwl / 2.0
        ply2 = pyl + phl / 2.0
        dist = jnp.sqrt((gcx - pxl) ** 2 + (gcy - pyl) ** 2)
        inter = (jnp.clip(jnp.minimum(bx2, plx2) - jnp.maximum(bx1, plx1),
                          0.0, None)
                 * jnp.clip(jnp.minimum(by2, ply2) - jnp.maximum(by1, ply1),
                            0.0, None))
        area_b = (plx2 - plx1) * (ply2 - ply1)
        ov = inter / (area_a + area_b - inter + 1e-10)

        loc_src = [
            jnp.concatenate(
                [jnp.broadcast_to(locs_ref[i, c:c + 1, s0:s1], (_N_OBJ, npl))
                 for i in range(nb)], axis=0)
            for c in range(4)]

        colio = _iota((nrows, npl), 1)
        gathered = [[] for _ in range(7)]
        for _j in range(_K):
            m = jnp.min(dist, axis=1, keepdims=True)
            idx = jnp.min(jnp.where(dist == m, colio, 2 ** 30),
                          axis=1, keepdims=True)
            hit = colio == idx

            def pick(x, hit=hit):
                xb = jnp.broadcast_to(x, hit.shape)
                return jnp.sum(jnp.where(hit, xb, 0.0), axis=1, keepdims=True)

            for dst, src in zip(gathered, (ov, pxl, pyl) + tuple(loc_src)):
                dst.append(pick(src))
            dist = jnp.where(hit, _BIG_F, dist)
        lvl.append(tuple(jnp.concatenate(g, axis=1) for g in gathered))

    def relayout(idx):
        cols = []
        for i in range(nb):
            for l in range(_N_LEVELS):
                cols.append(lvl[l][idx][8 * i:8 * i + 8, :])
        return jnp.concatenate(cols, axis=1)

    POV, PCX, PCY, GX, GY, GW, GH = (relayout(t) for t in range(7))

    thr_cols = []
    for i in range(nb):
        cat = POV[:, n45 * i:n45 * (i + 1)]
        mean = jnp.sum(cat, axis=1, keepdims=True) / n45
        var = jnp.sum((cat - mean) ** 2, axis=1, keepdims=True) / (n45 - 1)
        thr_cols.append(jnp.broadcast_to(mean + jnp.sqrt(var), (_N_OBJ, n45)))
    THR = jnp.concatenate(thr_cols, axis=1)

    def img_bcast(col):
        return jnp.concatenate(
            [jnp.broadcast_to(col[8 * i:8 * i + 8, :], (_N_OBJ, n45))
             for i in range(nb)], axis=1)

    BX1 = img_bcast(bx1)
    BY1 = img_bcast(by1)
    BX2 = img_bcast(bx2)
    BY2 = img_bcast(by2)
    lab_cols = [_lane_to_sublane(labels_ref[i:i + 1, :], _N_OBJ)
                .astype(jnp.float32) for i in range(nb)]
    LAB = jnp.concatenate(
        [jnp.broadcast_to(lab_cols[i], (_N_OBJ, n45)) for i in range(nb)],
        axis=1)

    inside = ((BX1 < PCX) & (PCX < BX2) & (BY1 < PCY) & (PCY < BY2))
    mask = (POV > THR) & inside
    val = jnp.where(mask, POV, 0.0)
    bv = jnp.max(val, axis=0, keepdims=True)
    rio = _iota((_N_OBJ, nlane), 0)
    bo = jnp.min(jnp.where(val == bv, rio, _N_OBJ), axis=0, keepdims=True)
    oh = rio == bo
    selp = (bv > 0.0).astype(jnp.float32)

    def rowpick(x):
        xb = jnp.broadcast_to(x, oh.shape)
        return jnp.sum(jnp.where(oh, xb, 0.0), axis=0, keepdims=True)

    labp = selp * rowpick(LAB)
    tlx1 = rowpick(BX1)
    tly1 = rowpick(BY1)
    tlx2 = rowpick(BX2)
    tly2 = rowpick(BY2)
    gx = rowpick(GX)
    gy = rowpick(GY)
    gw = rowpick(GW)
    gh = rowpick(GH)
    pcx = rowpick(PCX)
    pcy = rowpick(PCY)

    svec = jnp.concatenate(
        [jnp.full((1, _K), _SCALES[l], jnp.float32)
         for _ in range(nb) for l in range(_N_LEVELS)], axis=1)

    dcx = gx * svec / 10.0 + pcx
    dcy = gy * svec / 10.0 + pcy
    dw = jnp.exp(gw / 5.0) * svec
    dh = jnp.exp(gh / 5.0) * svec
    dlx1 = dcx - dw / 2.0
    dly1 = dcy - dh / 2.0
    dlx2 = dcx + dw / 2.0
    dly2 = dcy + dh / 2.0

    eps = 1e-7
    pw_ = dlx2 - dlx1
    ph_ = dly2 - dly1
    tw_ = tlx2 - tlx1
    th_ = tly2 - tly1
    iw = jnp.clip(jnp.minimum(dlx2, tlx2) - jnp.maximum(dlx1, tlx1), 0.0, None)
    ih = jnp.clip(jnp.minimum(dly2, tly2) - jnp.maximum(dly1, tly1), 0.0, None)
    inter = iw * ih
    union = pw_ * ph_ + tw_ * th_ - inter + eps
    iou = inter / union
    cw = jnp.maximum(dlx2, tlx2) - jnp.minimum(dlx1, tlx1)
    ch = jnp.maximum(dly2, tly2) - jnp.minimum(dly1, tly1)
    c2 = cw ** 2 + ch ** 2 + eps
    rho2 = ((dlx1 + dlx2 - tlx1 - tlx2) ** 2
            + (dly1 + dly2 - tly1 - tly2) ** 2) / 4.0
    v = (4.0 / (jnp.pi ** 2)) * (_atan_nonneg(tw_ / (th_ + eps))
                                 - _atan_nonneg(pw_ / (ph_ + eps))) ** 2
    a = v / (1.0 - iou + v + eps)
    per = 1.0 - (iou - rho2 / c2 - a * v)

    loc_num = jnp.sum(per * selp, axis=1, keepdims=True)
    sel_sum = jnp.sum(selp, axis=1, keepdims=True)
    npos = jnp.sum((labp > 0.0).astype(jnp.float32), axis=1, keepdims=True)

    bg = jnp.zeros((1, 1), jnp.float32)
    for i in range(nb):
        z = scores_ref[i]
        p, sp_pos, _ = _focal_terms(z)
        bg = bg + jnp.sum(0.75 * p * p * sp_pos)

    lab_sub = []
    for i in range(nb):
        row = jnp.concatenate(
            [labp[:, n45 * i:n45 * (i + 1)],
             jnp.zeros((1, 3), jnp.float32)], axis=1)
        lab_sub.append(_lane_to_sublane(row, 48))
    lab192 = jnp.concatenate(lab_sub, axis=0)

    zr = pos_ref[...]
    cio = _iota((zr.shape[0], _N_CLASSES), 1).astype(jnp.float32)
    tmask = (cio == lab192 - 1.0) & (lab192 > 0.0)
    pr, spr_pos, spr_neg = _focal_terms(zr)
    corr_terms = (0.25 * (1.0 - pr) ** 2 * spr_neg
                  - 0.75 * pr * pr * spr_pos)
    corr = jnp.sum(jnp.where(tmask, corr_terms, 0.0))

    focal = bg + corr
    conf = focal / jnp.maximum(npos, 1.0)
    locl = loc_num / jnp.maximum(sel_sum, 1.0)
    total = conf + locl

    oio = _iota((1, 128), 1)
    out_ref[...] = jnp.where(oio == 0, total, 0.0)


def _impl(predicted_locs, predicted_scores, boxes, labels, priors,
          interpret=False):
    batch = predicted_locs.shape[0]
    n_cls = predicted_scores.shape[2]

    locs_t = jnp.transpose(predicted_locs, (0, 2, 1))
    priors_t = jnp.transpose(priors, (1, 0))
    pos_rows = jnp.concatenate(
        [predicted_scores[:, s:s + _K, :] for s in _SPLITS[:-1]]
        + [jnp.zeros((batch, 3, n_cls), jnp.float32)], axis=1)
    pos2 = pos_rows.reshape(batch * 48, n_cls)
    boxes32 = boxes.reshape(batch * _N_OBJ, 4)

    parts = pl.pallas_call(
        _body,
        out_shape=jax.ShapeDtypeStruct((1, 128), jnp.float32),
        interpret=interpret,
    )(predicted_scores, pos2, locs_t, boxes32, labels, priors_t)
    return parts[0, 0]


def kernel(predicted_locs, predicted_scores, boxes, labels, priors):
    return _impl(predicted_locs, predicted_scores, boxes, labels, priors)


# speedup vs baseline: 1.0071x; 1.0071x over previous
import jax
import jax.numpy as jnp
from jax.experimental import pallas as pl

_SPLITS = (0, 6400, 8000, 8400, 8500, 8525)
_SCALES = (0.1, 0.2, 0.375, 0.55, 0.725)
_N_LEVELS = 5
_K = 9
_N_OBJ = 8
_N_CLASSES = 80
_BIG_F = 1e30

_L1P = (0.003662242172958829, -0.022628007027180385, 0.06573552525611641,
        -0.12447194531797226, 0.1842138633882333, -0.24618967713387793,
        0.3327853379900572, -0.4999589446838273, 0.9999987830866915,
        6.057848218605543e-09)


def _iota(shape, dim):
    return jax.lax.broadcasted_iota(jnp.int32, shape, dim)


def _lane_to_sublane(v, n):
    r = _iota((n, n), 0)
    c = _iota((n, n), 1)
    vb = jnp.broadcast_to(v, (n, n))
    zero = jnp.zeros((), v.dtype)
    return jnp.sum(jnp.where(r == c, vb, zero), axis=1, keepdims=True)


def _sublane_to_lane(v, n):
    r = _iota((n, n), 0)
    c = _iota((n, n), 1)
    vb = jnp.broadcast_to(v, (n, n))
    zero = jnp.zeros((), v.dtype)
    return jnp.sum(jnp.where(r == c, vb, zero), axis=0, keepdims=True)


def _log1p01(e):
    r = jnp.full_like(e, _L1P[0])
    for c in _L1P[1:]:
        r = r * e + c
    return r


def _focal_terms(x):
    e = jnp.exp(-jnp.abs(x))
    l1pe = _log1p01(e)
    rc = 1.0 / (1.0 + e)
    p = jnp.where(x >= 0.0, rc, e * rc)
    sp_pos = jnp.maximum(x, 0.0) + l1pe
    sp_neg = jnp.maximum(-x, 0.0) + l1pe
    return p, sp_pos, sp_neg


def _atan_nonneg(x):
    inv = x > 1.0
    y = jnp.where(inv, 1.0 / jnp.where(inv, x, 1.0), x)
    y = y / (1.0 + jnp.sqrt(1.0 + y * y))
    y = y / (1.0 + jnp.sqrt(1.0 + y * y))
    t = y * y
    s = y * (1.0 + t * (-1.0 / 3.0 + t * (1.0 / 5.0
                                          + t * (-1.0 / 7.0 + t / 9.0))))
    a = 4.0 * s
    return jnp.where(inv, jnp.pi / 2.0 - a, a)


def _assignment(pos_ref, locs_ref, boxes_ref, labels_ref, priors_ref):
    nb = locs_ref.shape[0]
    n45 = _N_LEVELS * _K
    nlane = nb * n45
    nrows = nb * _N_OBJ

    boxes = boxes_ref[...]
    bx1 = boxes[:, 0:1]
    by1 = boxes[:, 1:2]
    bx2 = boxes[:, 2:3]
    by2 = boxes[:, 3:4]
    gcx = (bx1 + bx2) * 0.5
    gcy = (by1 + by2) * 0.5
    area_a = (bx2 - bx1) * (by2 - by1)

    px = priors_ref[0:1, :]
    py = priors_ref[1:2, :]

    lvl = []
    for l in range(_N_LEVELS):
        s0, s1 = _SPLITS[l], _SPLITS[l + 1]
        npl = s1 - s0
        pxl = px[:, s0:s1]
        pyl = py[:, s0:s1]
        dist = jnp.sqrt((gcx - pxl) ** 2 + (gcy - pyl) ** 2)

        colio = _iota((nrows, npl), 1)
        pcx_j, pcy_j, ti_j = [], [], []
        for _j in range(_K):
            m = jnp.min(dist, axis=1, keepdims=True)
            idx = jnp.min(jnp.where(dist == m, colio, 2 ** 30),
                          axis=1, keepdims=True)
            hit = colio == idx
            pcx_j.append(jnp.sum(
                jnp.where(hit, jnp.broadcast_to(pxl, hit.shape), 0.0),
                axis=1, keepdims=True))
            pcy_j.append(jnp.sum(
                jnp.where(hit, jnp.broadcast_to(pyl, hit.shape), 0.0),
                axis=1, keepdims=True))
            ti_j.append(idx)
            dist = jnp.where(hit, _BIG_F, dist)
        pcx_l = jnp.concatenate(pcx_j, axis=1)
        pcy_l = jnp.concatenate(pcy_j, axis=1)
        ti_l = jnp.concatenate(ti_j, axis=1)

        half = _SCALES[l] / 2.0
        plx1 = pcx_l - half
        ply1 = pcy_l - half
        plx2 = pcx_l + half
        ply2 = pcy_l + half
        inter = (jnp.clip(jnp.minimum(bx2, plx2) - jnp.maximum(bx1, plx1),
                          0.0, None)
                 * jnp.clip(jnp.minimum(by2, ply2) - jnp.maximum(by1, ply1),
                            0.0, None))
        area_b = (plx2 - plx1) * (ply2 - ply1)
        ov_l = inter / (area_a + area_b - inter + 1e-10)
        lvl.append((ov_l, pcx_l, pcy_l, ti_l))

    def relayout(idx):
        cols = []
        for i in range(nb):
            for l in range(_N_LEVELS):
                cols.append(lvl[l][idx][8 * i:8 * i + 8, :])
        return jnp.concatenate(cols, axis=1)

    POV, PCX, PCY, TI = (relayout(t) for t in range(4))

    thr_cols = []
    for i in range(nb):
        cat = POV[:, n45 * i:n45 * (i + 1)]
        mean = jnp.sum(cat, axis=1, keepdims=True) / n45
        var = jnp.sum((cat - mean) ** 2, axis=1, keepdims=True) / (n45 - 1)
        thr_cols.append(jnp.broadcast_to(mean + jnp.sqrt(var), (_N_OBJ, n45)))
    THR = jnp.concatenate(thr_cols, axis=1)

    def img_bcast(col):
        return jnp.concatenate(
            [jnp.broadcast_to(col[8 * i:8 * i + 8, :], (_N_OBJ, n45))
             for i in range(nb)], axis=1)

    BX1 = img_bcast(bx1)
    BY1 = img_bcast(by1)
    BX2 = img_bcast(bx2)
    BY2 = img_bcast(by2)
    lab_cols = [_lane_to_sublane(labels_ref[i:i + 1, :], _N_OBJ)
                .astype(jnp.float32) for i in range(nb)]
    LAB = jnp.concatenate(
        [jnp.broadcast_to(lab_cols[i], (_N_OBJ, n45)) for i in range(nb)],
        axis=1)

    inside = ((BX1 < PCX) & (PCX < BX2) & (BY1 < PCY) & (PCY < BY2))
    mask = (POV > THR) & inside
    val = jnp.where(mask, POV, 0.0)
    bv = jnp.max(val, axis=0, keepdims=True)
    rio = _iota((_N_OBJ, nlane), 0)
    bo = jnp.min(jnp.where(val == bv, rio, _N_OBJ), axis=0, keepdims=True)
    oh = rio == bo
    selp = (bv > 0.0).astype(jnp.float32)

    def rowpick(x):
        xb = jnp.broadcast_to(x, oh.shape)
        return jnp.sum(jnp.where(oh, xb, 0.0), axis=0, keepdims=True)

    labp = selp * rowpick(LAB)
    tlx1 = rowpick(BX1)
    tly1 = rowpick(BY1)
    tlx2 = rowpick(BX2)
    tly2 = rowpick(BY2)
    pcx = rowpick(PCX)
    pcy = rowpick(PCY)
    pi = jnp.sum(jnp.where(oh, jnp.broadcast_to(TI, oh.shape), 0),
                 axis=0, keepdims=True)

    g_pieces = [[], [], [], []]
    for i in range(nb):
        for l in range(_N_LEVELS):
            s0, s1 = _SPLITS[l], _SPLITS[l + 1]
            npl = s1 - s0
            base = n45 * i + _K * l
            pi9 = pi[:, base:base + _K]
            pi9s = _lane_to_sublane(pi9, _K)
            ohp = _iota((_K, npl), 1) == pi9s
            for c in range(4):
                src = jnp.broadcast_to(locs_ref[i, c:c + 1, s0:s1],
                                       (_K, npl))
                v = jnp.sum(jnp.where(ohp, src, 0.0),
                            axis=1, keepdims=True)
                g_pieces[c].append(_sublane_to_lane(v, _K))
    gx, gy, gw, gh = (jnp.concatenate(p, axis=1) for p in g_pieces)

    svec = jnp.concatenate(
        [jnp.full((1, _K), _SCALES[l], jnp.float32)
         for _ in range(nb) for l in range(_N_LEVELS)], axis=1)

    dcx = gx * svec / 10.0 + pcx
    dcy = gy * svec / 10.0 + pcy
    dw = jnp.exp(gw / 5.0) * svec
    dh = jnp.exp(gh / 5.0) * svec
    dlx1 = dcx - dw / 2.0
    dly1 = dcy - dh / 2.0
    dlx2 = dcx + dw / 2.0
    dly2 = dcy + dh / 2.0

    eps = 1e-7
    pw_ = dlx2 - dlx1
    ph_ = dly2 - dly1
    tw_ = tlx2 - tlx1
    th_ = tly2 - tly1
    iw = jnp.clip(jnp.minimum(dlx2, tlx2) - jnp.maximum(dlx1, tlx1), 0.0, None)
    ih = jnp.clip(jnp.minimum(dly2, tly2) - jnp.maximum(dly1, tly1), 0.0, None)
    inter = iw * ih
    union = pw_ * ph_ + tw_ * th_ - inter + eps
    iou = inter / union
    cw = jnp.maximum(dlx2, tlx2) - jnp.minimum(dlx1, tlx1)
    ch = jnp.maximum(dly2, tly2) - jnp.minimum(dly1, tly1)
    c2 = cw ** 2 + ch ** 2 + eps
    rho2 = ((dlx1 + dlx2 - tlx1 - tlx2) ** 2
            + (dly1 + dly2 - tly1 - tly2) ** 2) / 4.0
    v = (4.0 / (jnp.pi ** 2)) * (_atan_nonneg(tw_ / (th_ + eps))
                                 - _atan_nonneg(pw_ / (ph_ + eps))) ** 2
    a = v / (1.0 - iou + v + eps)
    per = 1.0 - (iou - rho2 / c2 - a * v)

    loc_num = jnp.sum(per * selp, axis=1, keepdims=True)
    sel_sum = jnp.sum(selp, axis=1, keepdims=True)
    npos = jnp.sum((labp > 0.0).astype(jnp.float32), axis=1, keepdims=True)

    lab_sub = []
    for i in range(nb):
        row = jnp.concatenate(
            [labp[:, n45 * i:n45 * (i + 1)],
             jnp.zeros((1, 3), jnp.float32)], axis=1)
        lab_sub.append(_lane_to_sublane(row, 48))
    labc = jnp.concatenate(lab_sub, axis=0)

    zr = pos_ref[...]
    cio = _iota((zr.shape[0], _N_CLASSES), 1).astype(jnp.float32)
    tmask = (cio == labc - 1.0) & (labc > 0.0)
    pr, spr_pos, spr_neg = _focal_terms(zr)
    corr_terms = (0.25 * (1.0 - pr) ** 2 * spr_neg
                  - 0.75 * pr * pr * spr_pos)
    corr = jnp.sum(jnp.where(tmask, corr_terms, 0.0)).reshape(1, 1)

    return corr, npos, loc_num, sel_sum


def _body(scores_ref, pos_ref, locs_ref, boxes_ref, labels_ref, priors_ref,
          out_ref):
    step = pl.program_id(0)
    oio = _iota((1, 128), 1)

    z = scores_ref[0]
    p, sp_pos, _ = _focal_terms(z)
    bg = jnp.sum(0.75 * p * p * sp_pos)

    @pl.when(step == 0)
    def _first():
        corr, npos, loc_num, sel_sum = _assignment(
            pos_ref, locs_ref, boxes_ref, labels_ref, priors_ref)
        out_ref[...] = (jnp.where(oio == 0, bg + corr, 0.0)
                        + jnp.where(oio == 1, npos, 0.0)
                        + jnp.where(oio == 2, loc_num, 0.0)
                        + jnp.where(oio == 3, sel_sum, 0.0))

    @pl.when(step != 0)
    def _rest():
        out_ref[...] = out_ref[...] + jnp.where(oio == 0, bg, 0.0)


def _impl(predicted_locs, predicted_scores, boxes, labels, priors,
          interpret=False):
    batch = predicted_locs.shape[0]
    n_pri = predicted_locs.shape[1]
    n_cls = predicted_scores.shape[2]

    locs_t = jnp.transpose(predicted_locs, (0, 2, 1))
    priors_t = jnp.transpose(priors, (1, 0))
    pos_rows = jnp.concatenate(
        [predicted_scores[:, s:s + _K, :] for s in _SPLITS[:-1]]
        + [jnp.zeros((batch, 3, n_cls), jnp.float32)], axis=1)
    pos2 = pos_rows.reshape(batch * 48, n_cls)
    boxes32 = boxes.reshape(batch * _N_OBJ, 4)

    parts = pl.pallas_call(
        _body,
        grid=(batch,),
        in_specs=[
            pl.BlockSpec((1, n_pri, n_cls), lambda i: (i, 0, 0)),
            pl.BlockSpec((batch * 48, n_cls), lambda i: (0, 0)),
            pl.BlockSpec((batch, 4, n_pri), lambda i: (0, 0, 0)),
            pl.BlockSpec((batch * _N_OBJ, 4), lambda i: (0, 0)),
            pl.BlockSpec((batch, _N_OBJ), lambda i: (0, 0)),
            pl.BlockSpec((4, n_pri), lambda i: (0, 0)),
        ],
        out_specs=pl.BlockSpec((1, 128), lambda i: (0, 0)),
        out_shape=jax.ShapeDtypeStruct((1, 128), jnp.float32),
        interpret=interpret,
    )(predicted_scores, pos2, locs_t, boxes32, labels, priors_t)

    focal = parts[0, 0]
    npos = jnp.maximum(parts[0, 1], 1.0)
    loc_num = parts[0, 2]
    sel_sum = jnp.maximum(parts[0, 3], 1.0)
    return focal / npos + loc_num / sel_sum


def kernel(predicted_locs, predicted_scores, boxes, labels, priors):
    return _impl(predicted_locs, predicted_scores, boxes, labels, priors)


# speedup vs baseline: 1.6234x; 1.6120x over previous
import jax
import jax.numpy as jnp
from jax.experimental import pallas as pl

_SPLITS = (0, 6400, 8000, 8400, 8500, 8525)
_SCALES = (0.1, 0.2, 0.375, 0.55, 0.725)
_N_LEVELS = 5
_K = 9
_N_OBJ = 8
_N_CLASSES = 80
_BIG_F = 1e30

_L1P = (0.003662242172958829, -0.022628007027180385, 0.06573552525611641,
        -0.12447194531797226, 0.1842138633882333, -0.24618967713387793,
        0.3327853379900572, -0.4999589446838273, 0.9999987830866915,
        6.057848218605543e-09)


def _iota(shape, dim):
    return jax.lax.broadcasted_iota(jnp.int32, shape, dim)


def _lane_to_sublane(v, n):
    r = _iota((n, n), 0)
    c = _iota((n, n), 1)
    vb = jnp.broadcast_to(v, (n, n))
    zero = jnp.zeros((), v.dtype)
    return jnp.sum(jnp.where(r == c, vb, zero), axis=1, keepdims=True)


def _sublane_to_lane(v, n):
    r = _iota((n, n), 0)
    c = _iota((n, n), 1)
    vb = jnp.broadcast_to(v, (n, n))
    zero = jnp.zeros((), v.dtype)
    return jnp.sum(jnp.where(r == c, vb, zero), axis=0, keepdims=True)


def _focal_terms(x):
    t = jnp.tanh(0.5 * x)
    p = 0.5 + 0.5 * t
    q = 0.5 - 0.5 * t
    sp_pos = jnp.where(q > 0.0, -jnp.log(q), x)
    sp_neg = jnp.where(p > 0.0, -jnp.log(p), -x)
    return p, q, sp_pos, sp_neg


def _atan_nonneg(x):
    inv = x > 1.0
    y = jnp.where(inv, 1.0 / jnp.where(inv, x, 1.0), x)
    y = y / (1.0 + jnp.sqrt(1.0 + y * y))
    y = y / (1.0 + jnp.sqrt(1.0 + y * y))
    t = y * y
    s = y * (1.0 + t * (-1.0 / 3.0 + t * (1.0 / 5.0
                                          + t * (-1.0 / 7.0 + t / 9.0))))
    a = 4.0 * s
    return jnp.where(inv, jnp.pi / 2.0 - a, a)


def _assignment(pos_ref, locs_ref, boxes_ref, labels_ref, priors_ref):
    nb = locs_ref.shape[0]
    n45 = _N_LEVELS * _K
    nlane = nb * n45
    nrows = nb * _N_OBJ

    boxes = boxes_ref[...]
    bx1 = boxes[:, 0:1]
    by1 = boxes[:, 1:2]
    bx2 = boxes[:, 2:3]
    by2 = boxes[:, 3:4]
    gcx = (bx1 + bx2) * 0.5
    gcy = (by1 + by2) * 0.5
    area_a = (bx2 - bx1) * (by2 - by1)

    px = priors_ref[0:1, :]
    py = priors_ref[1:2, :]

    lvl = []
    for l in range(_N_LEVELS):
        s0, s1 = _SPLITS[l], _SPLITS[l + 1]
        npl = s1 - s0
        pxl = px[:, s0:s1]
        pyl = py[:, s0:s1]
        dist = jnp.sqrt((gcx - pxl) ** 2 + (gcy - pyl) ** 2)

        colio = _iota((nrows, npl), 1)
        pcx_j, pcy_j, ti_j = [], [], []
        for _j in range(_K):
            m = jnp.min(dist, axis=1, keepdims=True)
            idx = jnp.min(jnp.where(dist == m, colio, 2 ** 30),
                          axis=1, keepdims=True)
            hit = colio == idx
            pcx_j.append(jnp.sum(
                jnp.where(hit, jnp.broadcast_to(pxl, hit.shape), 0.0),
                axis=1, keepdims=True))
            pcy_j.append(jnp.sum(
                jnp.where(hit, jnp.broadcast_to(pyl, hit.shape), 0.0),
                axis=1, keepdims=True))
            ti_j.append(idx)
            dist = jnp.where(hit, _BIG_F, dist)
        pcx_l = jnp.concatenate(pcx_j, axis=1)
        pcy_l = jnp.concatenate(pcy_j, axis=1)
        ti_l = jnp.concatenate(ti_j, axis=1)

        half = _SCALES[l] / 2.0
        plx1 = pcx_l - half
        ply1 = pcy_l - half
        plx2 = pcx_l + half
        ply2 = pcy_l + half
        inter = (jnp.clip(jnp.minimum(bx2, plx2) - jnp.maximum(bx1, plx1),
                          0.0, None)
                 * jnp.clip(jnp.minimum(by2, ply2) - jnp.maximum(by1, ply1),
                            0.0, None))
        area_b = (plx2 - plx1) * (ply2 - ply1)
        ov_l = inter / (area_a + area_b - inter + 1e-10)
        lvl.append((ov_l, pcx_l, pcy_l, ti_l))

    def relayout(idx):
        cols = []
        for i in range(nb):
            for l in range(_N_LEVELS):
                cols.append(lvl[l][idx][8 * i:8 * i + 8, :])
        return jnp.concatenate(cols, axis=1)

    POV, PCX, PCY, TI = (relayout(t) for t in range(4))

    thr_cols = []
    for i in range(nb):
        cat = POV[:, n45 * i:n45 * (i + 1)]
        mean = jnp.sum(cat, axis=1, keepdims=True) / n45
        var = jnp.sum((cat - mean) ** 2, axis=1, keepdims=True) / (n45 - 1)
        thr_cols.append(jnp.broadcast_to(mean + jnp.sqrt(var), (_N_OBJ, n45)))
    THR = jnp.concatenate(thr_cols, axis=1)

    def img_bcast(col):
        return jnp.concatenate(
            [jnp.broadcast_to(col[8 * i:8 * i + 8, :], (_N_OBJ, n45))
             for i in range(nb)], axis=1)

    BX1 = img_bcast(bx1)
    BY1 = img_bcast(by1)
    BX2 = img_bcast(bx2)
    BY2 = img_bcast(by2)
    lab_cols = [_lane_to_sublane(labels_ref[i:i + 1, :], _N_OBJ)
                .astype(jnp.float32) for i in range(nb)]
    LAB = jnp.concatenate(
        [jnp.broadcast_to(lab_cols[i], (_N_OBJ, n45)) for i in range(nb)],
        axis=1)

    inside = ((BX1 < PCX) & (PCX < BX2) & (BY1 < PCY) & (PCY < BY2))
    mask = (POV > THR) & inside
    val = jnp.where(mask, POV, 0.0)
    bv = jnp.max(val, axis=0, keepdims=True)
    rio = _iota((_N_OBJ, nlane), 0)
    bo = jnp.min(jnp.where(val == bv, rio, _N_OBJ), axis=0, keepdims=True)
    oh = rio == bo
    selp = (bv > 0.0).astype(jnp.float32)

    def rowpick(x):
        xb = jnp.broadcast_to(x, oh.shape)
        return jnp.sum(jnp.where(oh, xb, 0.0), axis=0, keepdims=True)

    labp = selp * rowpick(LAB)
    tlx1 = rowpick(BX1)
    tly1 = rowpick(BY1)
    tlx2 = rowpick(BX2)
    tly2 = rowpick(BY2)
    pcx = rowpick(PCX)
    pcy = rowpick(PCY)
    pi = jnp.sum(jnp.where(oh, jnp.broadcast_to(TI, oh.shape), 0),
                 axis=0, keepdims=True)

    g_pieces = [[], [], [], []]
    for i in range(nb):
        for l in range(_N_LEVELS):
            s0, s1 = _SPLITS[l], _SPLITS[l + 1]
            npl = s1 - s0
            base = n45 * i + _K * l
            pi9 = pi[:, base:base + _K]
            pi9s = _lane_to_sublane(pi9, _K)
            ohp = _iota((_K, npl), 1) == pi9s
            for c in range(4):
                src = jnp.broadcast_to(locs_ref[i, c:c + 1, s0:s1],
                                       (_K, npl))
                v = jnp.sum(jnp.where(ohp, src, 0.0),
                            axis=1, keepdims=True)
                g_pieces[c].append(_sublane_to_lane(v, _K))
    gx, gy, gw, gh = (jnp.concatenate(p, axis=1) for p in g_pieces)

    svec = jnp.concatenate(
        [jnp.full((1, _K), _SCALES[l], jnp.float32)
         for _ in range(nb) for l in range(_N_LEVELS)], axis=1)

    dcx = gx * svec / 10.0 + pcx
    dcy = gy * svec / 10.0 + pcy
    dw = jnp.exp(gw / 5.0) * svec
    dh = jnp.exp(gh / 5.0) * svec
    dlx1 = dcx - dw / 2.0
    dly1 = dcy - dh / 2.0
    dlx2 = dcx + dw / 2.0
    dly2 = dcy + dh / 2.0

    eps = 1e-7
    pw_ = dlx2 - dlx1
    ph_ = dly2 - dly1
    tw_ = tlx2 - tlx1
    th_ = tly2 - tly1
    iw = jnp.clip(jnp.minimum(dlx2, tlx2) - jnp.maximum(dlx1, tlx1), 0.0, None)
    ih = jnp.clip(jnp.minimum(dly2, tly2) - jnp.maximum(dly1, tly1), 0.0, None)
    inter = iw * ih
    union = pw_ * ph_ + tw_ * th_ - inter + eps
    iou = inter / union
    cw = jnp.maximum(dlx2, tlx2) - jnp.minimum(dlx1, tlx1)
    ch = jnp.maximum(dly2, tly2) - jnp.minimum(dly1, tly1)
    c2 = cw ** 2 + ch ** 2 + eps
    rho2 = ((dlx1 + dlx2 - tlx1 - tlx2) ** 2
            + (dly1 + dly2 - tly1 - tly2) ** 2) / 4.0
    v = (4.0 / (jnp.pi ** 2)) * (_atan_nonneg(tw_ / (th_ + eps))
                                 - _atan_nonneg(pw_ / (ph_ + eps))) ** 2
    a = v / (1.0 - iou + v + eps)
    per = 1.0 - (iou - rho2 / c2 - a * v)

    loc_num = jnp.sum(per * selp, axis=1, keepdims=True)
    sel_sum = jnp.sum(selp, axis=1, keepdims=True)
    npos = jnp.sum((labp > 0.0).astype(jnp.float32), axis=1, keepdims=True)

    lab_sub = []
    for i in range(nb):
        row = jnp.concatenate(
            [labp[:, n45 * i:n45 * (i + 1)],
             jnp.zeros((1, 3), jnp.float32)], axis=1)
        lab_sub.append(_lane_to_sublane(row, 48))
    labc = jnp.concatenate(lab_sub, axis=0)

    zr = pos_ref[...]
    cio = _iota((zr.shape[0], _N_CLASSES), 1).astype(jnp.float32)
    tmask = (cio == labc - 1.0) & (labc > 0.0)
    pr, qr, spr_pos, spr_neg = _focal_terms(zr)
    corr_terms = (0.25 * qr * qr * spr_neg
                  - 0.75 * pr * pr * spr_pos)
    corr = jnp.sum(jnp.where(tmask, corr_terms, 0.0)).reshape(1, 1)

    return corr, npos, loc_num, sel_sum


def _body(scores_ref, pos_ref, locs_ref, boxes_ref, labels_ref, priors_ref,
          out_ref):
    step = pl.program_id(0)
    oio = _iota((1, 128), 1)

    z = scores_ref[0]
    p, _, sp_pos, _ = _focal_terms(z)
    bg = 0.75 * jnp.sum(p * p * sp_pos)

    @pl.when(step == 0)
    def _first():
        corr, npos, loc_num, sel_sum = _assignment(
            pos_ref, locs_ref, boxes_ref, labels_ref, priors_ref)
        out_ref[...] = (jnp.where(oio == 0, bg + corr, 0.0)
                        + jnp.where(oio == 1, npos, 0.0)
                        + jnp.where(oio == 2, loc_num, 0.0)
                        + jnp.where(oio == 3, sel_sum, 0.0))

    @pl.when(step != 0)
    def _rest():
        out_ref[...] = out_ref[...] + jnp.where(oio == 0, bg, 0.0)


def _impl(predicted_locs, predicted_scores, boxes, labels, priors,
          interpret=False):
    batch = predicted_locs.shape[0]
    n_pri = predicted_locs.shape[1]
    n_cls = predicted_scores.shape[2]

    locs_t = jnp.transpose(predicted_locs, (0, 2, 1))
    priors_t = jnp.transpose(priors, (1, 0))
    pos_rows = jnp.concatenate(
        [predicted_scores[:, s:s + _K, :] for s in _SPLITS[:-1]]
        + [jnp.zeros((batch, 3, n_cls), jnp.float32)], axis=1)
    pos2 = pos_rows.reshape(batch * 48, n_cls)
    boxes32 = boxes.reshape(batch * _N_OBJ, 4)

    parts = pl.pallas_call(
        _body,
        grid=(batch,),
        in_specs=[
            pl.BlockSpec((1, n_pri, n_cls), lambda i: (i, 0, 0)),
            pl.BlockSpec((batch * 48, n_cls), lambda i: (0, 0)),
            pl.BlockSpec((batch, 4, n_pri), lambda i: (0, 0, 0)),
            pl.BlockSpec((batch * _N_OBJ, 4), lambda i: (0, 0)),
            pl.BlockSpec((batch, _N_OBJ), lambda i: (0, 0)),
            pl.BlockSpec((4, n_pri), lambda i: (0, 0)),
        ],
        out_specs=pl.BlockSpec((1, 128), lambda i: (0, 0)),
        out_shape=jax.ShapeDtypeStruct((1, 128), jnp.float32),
        interpret=interpret,
    )(predicted_scores, pos2, locs_t, boxes32, labels, priors_t)

    focal = parts[0, 0]
    npos = jnp.maximum(parts[0, 1], 1.0)
    loc_num = parts[0, 2]
    sel_sum = jnp.maximum(parts[0, 3], 1.0)
    return focal / npos + loc_num / sel_sum


def kernel(predicted_locs, predicted_scores, boxes, labels, priors):
    return _impl(predicted_locs, predicted_scores, boxes, labels, priors)


# speedup vs baseline: 1.8230x; 1.1230x over previous
import jax
import jax.numpy as jnp
from jax.experimental import pallas as pl
from jax.experimental.pallas import tpu as pltpu

_FMAPS = (80, 40, 20, 10, 5)
_SPLITS = (0, 6400, 8000, 8400, 8500, 8525)
_SCALES = (0.1, 0.2, 0.375, 0.55, 0.725)
_N_LEVELS = 5
_K = 9
_N_OBJ = 8
_N_CLASSES = 80
_BIG_F = 1e30

_L1P = (0.003662242172958829, -0.022628007027180385, 0.06573552525611641,
        -0.12447194531797226, 0.1842138633882333, -0.24618967713387793,
        0.3327853379900572, -0.4999589446838273, 0.9999987830866915,
        6.057848218605543e-09)


def _iota(shape, dim):
    return jax.lax.broadcasted_iota(jnp.int32, shape, dim)


def _lane_to_sublane(v, n):
    r = _iota((n, n), 0)
    c = _iota((n, n), 1)
    vb = jnp.broadcast_to(v, (n, n))
    zero = jnp.zeros((), v.dtype)
    return jnp.sum(jnp.where(r == c, vb, zero), axis=1, keepdims=True)


def _sublane_to_lane(v, n):
    r = _iota((n, n), 0)
    c = _iota((n, n), 1)
    vb = jnp.broadcast_to(v, (n, n))
    zero = jnp.zeros((), v.dtype)
    return jnp.sum(jnp.where(r == c, vb, zero), axis=0, keepdims=True)


def _focal_terms(x):
    t = jnp.tanh(0.5 * x)
    p = 0.5 + 0.5 * t
    q = 0.5 - 0.5 * t
    sp_pos = jnp.where(q > 0.0, -jnp.log(q), x)
    sp_neg = jnp.where(p > 0.0, -jnp.log(p), -x)
    return p, q, sp_pos, sp_neg


def _atan_nonneg(x):
    inv = x > 1.0
    y = jnp.where(inv, 1.0 / jnp.where(inv, x, 1.0), x)
    y = y / (1.0 + jnp.sqrt(1.0 + y * y))
    y = y / (1.0 + jnp.sqrt(1.0 + y * y))
    t = y * y
    s = y * (1.0 + t * (-1.0 / 3.0 + t * (1.0 / 5.0
                                          + t * (-1.0 / 7.0 + t / 9.0))))
    a = 4.0 * s
    return jnp.where(inv, jnp.pi / 2.0 - a, a)


def _assignment(pos_ref, locs_ref, boxes_ref, labels_ref, priors_ref,
                boxes_sm_ref):
    nb = locs_ref.shape[0]
    n45 = _N_LEVELS * _K
    nlane = nb * n45
    nrows = nb * _N_OBJ

    boxes = boxes_ref[...]
    bx1 = boxes[:, 0:1]
    by1 = boxes[:, 1:2]
    bx2 = boxes[:, 2:3]
    by2 = boxes[:, 3:4]
    gcx = (bx1 + bx2) * 0.5
    gcy = (by1 + by2) * 0.5
    area_a = (bx2 - bx1) * (by2 - by1)

    px = priors_ref[0:1, :]
    py = priors_ref[1:2, :]

    lvl = []
    for l in range(_N_LEVELS):
        s0, s1 = _SPLITS[l], _SPLITS[l + 1]
        f = _FMAPS[l]
        if l <= 1:
            npl = 896 if l == 0 else 512
            lvl_len = s1 - s0
            starts, limits = [], []
            for r in range(nrows):
                qy = (boxes_sm_ref[r, 1] + boxes_sm_ref[r, 3]) * 0.5
                fr = jnp.floor(qy * f - 0.5).astype(jnp.int32)
                band0 = jnp.clip(fr - 4, 0, f - 9) * f
                starts.append((band0 // 128) * 128)
                limits.append(lvl_len - starts[-1])
            pxl = jnp.concatenate(
                [priors_ref[0:1,
                            pl.ds(pl.multiple_of(s0 + starts[r], 128), npl)]
                 for r in range(nrows)], axis=0)
            pyl = jnp.concatenate(
                [priors_ref[1:2,
                            pl.ds(pl.multiple_of(s0 + starts[r], 128), npl)]
                 for r in range(nrows)], axis=0)
            off_col = jnp.concatenate(
                [jnp.full((1, 1), starts[r], jnp.int32)
                 for r in range(nrows)], axis=0)
            limit_col = jnp.concatenate(
                [jnp.full((1, 1), limits[r], jnp.int32)
                 for r in range(nrows)], axis=0)
        else:
            npl = s1 - s0
            pxl = px[:, s0:s1]
            pyl = py[:, s0:s1]
            off_col = jnp.zeros((nrows, 1), jnp.int32)
            limit_col = None
        dist = jnp.sqrt((gcx - pxl) ** 2 + (gcy - pyl) ** 2)
        if limit_col is not None:
            dist = jnp.where(_iota((nrows, npl), 1) < limit_col,
                             dist, _BIG_F)

        colio = _iota((nrows, npl), 1)
        pcx_j, pcy_j, ti_j = [], [], []
        for _j in range(_K):
            m = jnp.min(dist, axis=1, keepdims=True)
            idx = jnp.min(jnp.where(dist == m, colio, 2 ** 30),
                          axis=1, keepdims=True)
            hit = colio == idx
            pcx_j.append(jnp.sum(
                jnp.where(hit, jnp.broadcast_to(pxl, hit.shape), 0.0),
                axis=1, keepdims=True))
            pcy_j.append(jnp.sum(
                jnp.where(hit, jnp.broadcast_to(pyl, hit.shape), 0.0),
                axis=1, keepdims=True))
            ti_j.append(idx)
            dist = jnp.where(hit, _BIG_F, dist)
        pcx_l = jnp.concatenate(pcx_j, axis=1)
        pcy_l = jnp.concatenate(pcy_j, axis=1)
        ti_l = jnp.concatenate(ti_j, axis=1) + off_col

        half = _SCALES[l] / 2.0
        plx1 = pcx_l - half
        ply1 = pcy_l - half
        plx2 = pcx_l + half
        ply2 = pcy_l + half
        inter = (jnp.clip(jnp.minimum(bx2, plx2) - jnp.maximum(bx1, plx1),
                          0.0, None)
                 * jnp.clip(jnp.minimum(by2, ply2) - jnp.maximum(by1, ply1),
                            0.0, None))
        area_b = (plx2 - plx1) * (ply2 - ply1)
        ov_l = inter / (area_a + area_b - inter + 1e-10)
        lvl.append((ov_l, pcx_l, pcy_l, ti_l))

    def relayout(idx):
        cols = []
        for i in range(nb):
            for l in range(_N_LEVELS):
                cols.append(lvl[l][idx][8 * i:8 * i + 8, :])
        return jnp.concatenate(cols, axis=1)

    POV, PCX, PCY, TI = (relayout(t) for t in range(4))

    thr_cols = []
    for i in range(nb):
        cat = POV[:, n45 * i:n45 * (i + 1)]
        mean = jnp.sum(cat, axis=1, keepdims=True) / n45
        var = jnp.sum((cat - mean) ** 2, axis=1, keepdims=True) / (n45 - 1)
        thr_cols.append(jnp.broadcast_to(mean + jnp.sqrt(var), (_N_OBJ, n45)))
    THR = jnp.concatenate(thr_cols, axis=1)

    def img_bcast(col):
        return jnp.concatenate(
            [jnp.broadcast_to(col[8 * i:8 * i + 8, :], (_N_OBJ, n45))
             for i in range(nb)], axis=1)

    BX1 = img_bcast(bx1)
    BY1 = img_bcast(by1)
    BX2 = img_bcast(bx2)
    BY2 = img_bcast(by2)
    lab_cols = [_lane_to_sublane(labels_ref[i:i + 1, :], _N_OBJ)
                .astype(jnp.float32) for i in range(nb)]
    LAB = jnp.concatenate(
        [jnp.broadcast_to(lab_cols[i], (_N_OBJ, n45)) for i in range(nb)],
        axis=1)

    inside = ((BX1 < PCX) & (PCX < BX2) & (BY1 < PCY) & (PCY < BY2))
    mask = (POV > THR) & inside
    val = jnp.where(mask, POV, 0.0)
    bv = jnp.max(val, axis=0, keepdims=True)
    rio = _iota((_N_OBJ, nlane), 0)
    bo = jnp.min(jnp.where(val == bv, rio, _N_OBJ), axis=0, keepdims=True)
    oh = rio == bo
    selp = (bv > 0.0).astype(jnp.float32)

    def rowpick(x):
        xb = jnp.broadcast_to(x, oh.shape)
        return jnp.sum(jnp.where(oh, xb, 0.0), axis=0, keepdims=True)

    labp = selp * rowpick(LAB)
    tlx1 = rowpick(BX1)
    tly1 = rowpick(BY1)
    tlx2 = rowpick(BX2)
    tly2 = rowpick(BY2)
    pcx = rowpick(PCX)
    pcy = rowpick(PCY)
    pi = jnp.sum(jnp.where(oh, jnp.broadcast_to(TI, oh.shape), 0),
                 axis=0, keepdims=True)

    g_pieces = [[], [], [], []]
    for i in range(nb):
        for l in range(_N_LEVELS):
            s0, s1 = _SPLITS[l], _SPLITS[l + 1]
            npl = s1 - s0
            base = n45 * i + _K * l
            pi9 = pi[:, base:base + _K]
            pi9s = _lane_to_sublane(pi9, _K)
            ohp = _iota((_K, npl), 1) == pi9s
            for c in range(4):
                src = jnp.broadcast_to(locs_ref[i, c:c + 1, s0:s1],
                                       (_K, npl))
                v = jnp.sum(jnp.where(ohp, src, 0.0),
                            axis=1, keepdims=True)
                g_pieces[c].append(_sublane_to_lane(v, _K))
    gx, gy, gw, gh = (jnp.concatenate(p, axis=1) for p in g_pieces)

    svec = jnp.concatenate(
        [jnp.full((1, _K), _SCALES[l], jnp.float32)
         for _ in range(nb) for l in range(_N_LEVELS)], axis=1)

    dcx = gx * svec / 10.0 + pcx
    dcy = gy * svec / 10.0 + pcy
    dw = jnp.exp(gw / 5.0) * svec
    dh = jnp.exp(gh / 5.0) * svec
    dlx1 = dcx - dw / 2.0
    dly1 = dcy - dh / 2.0
    dlx2 = dcx + dw / 2.0
    dly2 = dcy + dh / 2.0

    eps = 1e-7
    pw_ = dlx2 - dlx1
    ph_ = dly2 - dly1
    tw_ = tlx2 - tlx1
    th_ = tly2 - tly1
    iw = jnp.clip(jnp.minimum(dlx2, tlx2) - jnp.maximum(dlx1, tlx1), 0.0, None)
    ih = jnp.clip(jnp.minimum(dly2, tly2) - jnp.maximum(dly1, tly1), 0.0, None)
    inter = iw * ih
    union = pw_ * ph_ + tw_ * th_ - inter + eps
    iou = inter / union
    cw = jnp.maximum(dlx2, tlx2) - jnp.minimum(dlx1, tlx1)
    ch = jnp.maximum(dly2, tly2) - jnp.minimum(dly1, tly1)
    c2 = cw ** 2 + ch ** 2 + eps
    rho2 = ((dlx1 + dlx2 - tlx1 - tlx2) ** 2
            + (dly1 + dly2 - tly1 - tly2) ** 2) / 4.0
    v = (4.0 / (jnp.pi ** 2)) * (_atan_nonneg(tw_ / (th_ + eps))
                                 - _atan_nonneg(pw_ / (ph_ + eps))) ** 2
    a = v / (1.0 - iou + v + eps)
    per = 1.0 - (iou - rho2 / c2 - a * v)

    loc_num = jnp.sum(per * selp, axis=1, keepdims=True)
    sel_sum = jnp.sum(selp, axis=1, keepdims=True)
    npos = jnp.sum((labp > 0.0).astype(jnp.float32), axis=1, keepdims=True)

    lab_sub = []
    for i in range(nb):
        row = jnp.concatenate(
            [labp[:, n45 * i:n45 * (i + 1)],
             jnp.zeros((1, 3), jnp.float32)], axis=1)
        lab_sub.append(_lane_to_sublane(row, 48))
    labc = jnp.concatenate(lab_sub, axis=0)

    zr = pos_ref[...]
    cio = _iota((zr.shape[0], _N_CLASSES), 1).astype(jnp.float32)
    tmask = (cio == labc - 1.0) & (labc > 0.0)
    pr, qr, spr_pos, spr_neg = _focal_terms(zr)
    corr_terms = (0.25 * qr * qr * spr_neg
                  - 0.75 * pr * pr * spr_pos)
    corr = jnp.sum(jnp.where(tmask, corr_terms, 0.0)).reshape(1, 1)

    return corr, npos, loc_num, sel_sum


def _body(scores_ref, pos_ref, locs_ref, boxes_ref, labels_ref, priors_ref,
          boxes_sm_ref, out_ref):
    step = pl.program_id(0)
    oio = _iota((1, 128), 1)

    z = scores_ref[0]
    p, _, sp_pos, _ = _focal_terms(z)
    bg = 0.75 * jnp.sum(p * p * sp_pos)

    @pl.when(step == 0)
    def _first():
        corr, npos, loc_num, sel_sum = _assignment(
            pos_ref, locs_ref, boxes_ref, labels_ref, priors_ref,
            boxes_sm_ref)
        out_ref[...] = (jnp.where(oio == 0, bg + corr, 0.0)
                        + jnp.where(oio == 1, npos, 0.0)
                        + jnp.where(oio == 2, loc_num, 0.0)
                        + jnp.where(oio == 3, sel_sum, 0.0))

    @pl.when(step != 0)
    def _rest():
        out_ref[...] = out_ref[...] + jnp.where(oio == 0, bg, 0.0)


def _impl(predicted_locs, predicted_scores, boxes, labels, priors,
          interpret=False):
    batch = predicted_locs.shape[0]
    n_pri = predicted_locs.shape[1]
    n_cls = predicted_scores.shape[2]

    locs_t = jnp.transpose(predicted_locs, (0, 2, 1))
    priors_t = jnp.transpose(priors, (1, 0))
    pos_rows = jnp.concatenate(
        [predicted_scores[:, s:s + _K, :] for s in _SPLITS[:-1]]
        + [jnp.zeros((batch, 3, n_cls), jnp.float32)], axis=1)
    pos2 = pos_rows.reshape(batch * 48, n_cls)
    boxes32 = boxes.reshape(batch * _N_OBJ, 4)

    parts = pl.pallas_call(
        _body,
        grid=(batch,),
        in_specs=[
            pl.BlockSpec((1, n_pri, n_cls), lambda i: (i, 0, 0)),
            pl.BlockSpec((batch * 48, n_cls), lambda i: (0, 0)),
            pl.BlockSpec((batch, 4, n_pri), lambda i: (0, 0, 0)),
            pl.BlockSpec((batch * _N_OBJ, 4), lambda i: (0, 0)),
            pl.BlockSpec((batch, _N_OBJ), lambda i: (0, 0)),
            pl.BlockSpec((4, n_pri), lambda i: (0, 0)),
            pl.BlockSpec(memory_space=pltpu.SMEM),
        ],
        out_specs=pl.BlockSpec((1, 128), lambda i: (0, 0)),
        out_shape=jax.ShapeDtypeStruct((1, 128), jnp.float32),
        interpret=interpret,
    )(predicted_scores, pos2, locs_t, boxes32, labels, priors_t, boxes32)

    focal = parts[0, 0]
    npos = jnp.maximum(parts[0, 1], 1.0)
    loc_num = parts[0, 2]
    sel_sum = jnp.maximum(parts[0, 3], 1.0)
    return focal / npos + loc_num / sel_sum


def kernel(predicted_locs, predicted_scores, boxes, labels, priors):
    return _impl(predicted_locs, predicted_scores, boxes, labels, priors)


# speedup vs baseline: 1.8232x; 1.0001x over previous
import jax
import jax.numpy as jnp
from jax.experimental import pallas as pl
from jax.experimental.pallas import tpu as pltpu

_FMAPS = (80, 40, 20, 10, 5)
_SPLITS = (0, 6400, 8000, 8400, 8500, 8525)
_SCALES = (0.1, 0.2, 0.375, 0.55, 0.725)
_N_LEVELS = 5
_K = 9
_N_OBJ = 8
_N_CLASSES = 80
_BIG_F = 1e30


def _iota(shape, dim):
    return jax.lax.broadcasted_iota(jnp.int32, shape, dim)


def _lane_to_sublane(v, n):
    r = _iota((n, n), 0)
    c = _iota((n, n), 1)
    vb = jnp.broadcast_to(v, (n, n))
    zero = jnp.zeros((), v.dtype)
    return jnp.sum(jnp.where(r == c, vb, zero), axis=1, keepdims=True)


def _sublane_to_lane(v, n):
    r = _iota((n, n), 0)
    c = _iota((n, n), 1)
    vb = jnp.broadcast_to(v, (n, n))
    zero = jnp.zeros((), v.dtype)
    return jnp.sum(jnp.where(r == c, vb, zero), axis=0, keepdims=True)


def _focal_terms(x):
    t = jnp.tanh(0.5 * x)
    p = 0.5 + 0.5 * t
    q = 0.5 - 0.5 * t
    sp_pos = jnp.where(q > 0.0, -jnp.log(q), x)
    sp_neg = jnp.where(p > 0.0, -jnp.log(p), -x)
    return p, q, sp_pos, sp_neg


def _atan_nonneg(x):
    inv = x > 1.0
    y = jnp.where(inv, 1.0 / jnp.where(inv, x, 1.0), x)
    y = y / (1.0 + jnp.sqrt(1.0 + y * y))
    y = y / (1.0 + jnp.sqrt(1.0 + y * y))
    t = y * y
    s = y * (1.0 + t * (-1.0 / 3.0 + t * (1.0 / 5.0
                                          + t * (-1.0 / 7.0 + t / 9.0))))
    a = 4.0 * s
    return jnp.where(inv, jnp.pi / 2.0 - a, a)


def _assignment(pos_ref, locs_ref, boxes_ref, labels_ref, priors_ref,
                boxes_sm_ref):
    nb = locs_ref.shape[0]
    n45 = _N_LEVELS * _K
    nlane = nb * n45
    nrows = nb * _N_OBJ

    boxes = boxes_ref[...]
    bx1 = boxes[:, 0:1]
    by1 = boxes[:, 1:2]
    bx2 = boxes[:, 2:3]
    by2 = boxes[:, 3:4]
    gcx = (bx1 + bx2) * 0.5
    gcy = (by1 + by2) * 0.5
    area_a = (bx2 - bx1) * (by2 - by1)

    px = priors_ref[0:1, :]
    py = priors_ref[1:2, :]

    lvl = []
    for l in range(_N_LEVELS):
        s0, s1 = _SPLITS[l], _SPLITS[l + 1]
        f = _FMAPS[l]
        if l <= 1:
            npl = 896 if l == 0 else 512
            lvl_len = s1 - s0
            starts, limits = [], []
            for r in range(nrows):
                qy = (boxes_sm_ref[r, 1] + boxes_sm_ref[r, 3]) * 0.5
                fr = jnp.floor(qy * f - 0.5).astype(jnp.int32)
                band0 = jnp.clip(fr - 4, 0, f - 9) * f
                starts.append((band0 // 128) * 128)
                limits.append(lvl_len - starts[-1])
            pxl = jnp.concatenate(
                [priors_ref[0:1,
                            pl.ds(pl.multiple_of(s0 + starts[r], 128), npl)]
                 for r in range(nrows)], axis=0)
            pyl = jnp.concatenate(
                [priors_ref[1:2,
                            pl.ds(pl.multiple_of(s0 + starts[r], 128), npl)]
                 for r in range(nrows)], axis=0)
            off_col = jnp.concatenate(
                [jnp.full((1, 1), starts[r], jnp.int32)
                 for r in range(nrows)], axis=0)
            limit_col = jnp.concatenate(
                [jnp.full((1, 1), limits[r], jnp.int32)
                 for r in range(nrows)], axis=0)
        else:
            npl = s1 - s0
            pxl = px[:, s0:s1]
            pyl = py[:, s0:s1]
            off_col = jnp.zeros((nrows, 1), jnp.int32)
            limit_col = None
        dist = jnp.sqrt((gcx - pxl) ** 2 + (gcy - pyl) ** 2)
        if limit_col is not None:
            dist = jnp.where(_iota((nrows, npl), 1) < limit_col,
                             dist, _BIG_F)

        colio = _iota((nrows, npl), 1)
        pcx_j, pcy_j, ti_j = [], [], []
        for _j in range(_K):
            m = jnp.min(dist, axis=1, keepdims=True)
            idx = jnp.min(jnp.where(dist == m, colio, 2 ** 30),
                          axis=1, keepdims=True)
            hit = colio == idx
            pcx_j.append(jnp.sum(
                jnp.where(hit, jnp.broadcast_to(pxl, hit.shape), 0.0),
                axis=1, keepdims=True))
            pcy_j.append(jnp.sum(
                jnp.where(hit, jnp.broadcast_to(pyl, hit.shape), 0.0),
                axis=1, keepdims=True))
            ti_j.append(idx)
            dist = jnp.where(hit, _BIG_F, dist)
        pcx_l = jnp.concatenate(pcx_j, axis=1)
        pcy_l = jnp.concatenate(pcy_j, axis=1)
        ti_l = jnp.concatenate(ti_j, axis=1) + off_col

        half = _SCALES[l] / 2.0
        plx1 = pcx_l - half
        ply1 = pcy_l - half
        plx2 = pcx_l + half
        ply2 = pcy_l + half
        inter = (jnp.clip(jnp.minimum(bx2, plx2) - jnp.maximum(bx1, plx1),
                          0.0, None)
                 * jnp.clip(jnp.minimum(by2, ply2) - jnp.maximum(by1, ply1),
                            0.0, None))
        area_b = (plx2 - plx1) * (ply2 - ply1)
        ov_l = inter / (area_a + area_b - inter + 1e-10)
        lvl.append((ov_l, pcx_l, pcy_l, ti_l))

    def relayout(idx):
        cols = []
        for i in range(nb):
            for l in range(_N_LEVELS):
                cols.append(lvl[l][idx][8 * i:8 * i + 8, :])
        return jnp.concatenate(cols, axis=1)

    POV, PCX, PCY, TI = (relayout(t) for t in range(4))

    thr_cols = []
    for i in range(nb):
        cat = POV[:, n45 * i:n45 * (i + 1)]
        mean = jnp.sum(cat, axis=1, keepdims=True) / n45
        var = jnp.sum((cat - mean) ** 2, axis=1, keepdims=True) / (n45 - 1)
        thr_cols.append(jnp.broadcast_to(mean + jnp.sqrt(var), (_N_OBJ, n45)))
    THR = jnp.concatenate(thr_cols, axis=1)

    def img_bcast(col):
        return jnp.concatenate(
            [jnp.broadcast_to(col[8 * i:8 * i + 8, :], (_N_OBJ, n45))
             for i in range(nb)], axis=1)

    BX1 = img_bcast(bx1)
    BY1 = img_bcast(by1)
    BX2 = img_bcast(bx2)
    BY2 = img_bcast(by2)
    lab_cols = [_lane_to_sublane(labels_ref[i:i + 1, :], _N_OBJ)
                .astype(jnp.float32) for i in range(nb)]
    LAB = jnp.concatenate(
        [jnp.broadcast_to(lab_cols[i], (_N_OBJ, n45)) for i in range(nb)],
        axis=1)

    inside = ((BX1 < PCX) & (PCX < BX2) & (BY1 < PCY) & (PCY < BY2))
    mask = (POV > THR) & inside
    val = jnp.where(mask, POV, 0.0)
    bv = jnp.max(val, axis=0, keepdims=True)
    rio = _iota((_N_OBJ, nlane), 0)
    bo = jnp.min(jnp.where(val == bv, rio, _N_OBJ), axis=0, keepdims=True)
    oh = rio == bo
    selp = (bv > 0.0).astype(jnp.float32)

    def rowpick(x):
        xb = jnp.broadcast_to(x, oh.shape)
        return jnp.sum(jnp.where(oh, xb, 0.0), axis=0, keepdims=True)

    labp = selp * rowpick(LAB)
    tlx1 = rowpick(BX1)
    tly1 = rowpick(BY1)
    tlx2 = rowpick(BX2)
    tly2 = rowpick(BY2)
    pcx = rowpick(PCX)
    pcy = rowpick(PCY)
    pi = jnp.sum(jnp.where(oh, jnp.broadcast_to(TI, oh.shape), 0),
                 axis=0, keepdims=True)

    g_pieces = [[], [], [], []]
    for i in range(nb):
        for l in range(_N_LEVELS):
            s0, s1 = _SPLITS[l], _SPLITS[l + 1]
            npl = s1 - s0
            base = n45 * i + _K * l
            pi9 = pi[:, base:base + _K]
            pi9s = _lane_to_sublane(pi9, _K)
            ohp = _iota((_K, npl), 1) == pi9s
            for c in range(4):
                src = jnp.broadcast_to(locs_ref[i, c:c + 1, s0:s1],
                                       (_K, npl))
                v = jnp.sum(jnp.where(ohp, src, 0.0),
                            axis=1, keepdims=True)
                g_pieces[c].append(_sublane_to_lane(v, _K))
    gx, gy, gw, gh = (jnp.concatenate(p, axis=1) for p in g_pieces)

    svec = jnp.concatenate(
        [jnp.full((1, _K), _SCALES[l], jnp.float32)
         for _ in range(nb) for l in range(_N_LEVELS)], axis=1)

    dcx = gx * svec / 10.0 + pcx
    dcy = gy * svec / 10.0 + pcy
    dw = jnp.exp(gw / 5.0) * svec
    dh = jnp.exp(gh / 5.0) * svec
    dlx1 = dcx - dw / 2.0
    dly1 = dcy - dh / 2.0
    dlx2 = dcx + dw / 2.0
    dly2 = dcy + dh / 2.0

    eps = 1e-7
    pw_ = dlx2 - dlx1
    ph_ = dly2 - dly1
    tw_ = tlx2 - tlx1
    th_ = tly2 - tly1
    iw = jnp.clip(jnp.minimum(dlx2, tlx2) - jnp.maximum(dlx1, tlx1), 0.0, None)
    ih = jnp.clip(jnp.minimum(dly2, tly2) - jnp.maximum(dly1, tly1), 0.0, None)
    inter = iw * ih
    union = pw_ * ph_ + tw_ * th_ - inter + eps
    iou = inter / union
    cw = jnp.maximum(dlx2, tlx2) - jnp.minimum(dlx1, tlx1)
    ch = jnp.maximum(dly2, tly2) - jnp.minimum(dly1, tly1)
    c2 = cw ** 2 + ch ** 2 + eps
    rho2 = ((dlx1 + dlx2 - tlx1 - tlx2) ** 2
            + (dly1 + dly2 - tly1 - tly2) ** 2) / 4.0
    v = (4.0 / (jnp.pi ** 2)) * (_atan_nonneg(tw_ / (th_ + eps))
                                 - _atan_nonneg(pw_ / (ph_ + eps))) ** 2
    a = v / (1.0 - iou + v + eps)
    per = 1.0 - (iou - rho2 / c2 - a * v)

    loc_num = jnp.sum(per * selp, axis=1, keepdims=True)
    sel_sum = jnp.sum(selp, axis=1, keepdims=True)
    npos = jnp.sum((labp > 0.0).astype(jnp.float32), axis=1, keepdims=True)

    lab_sub = []
    for i in range(nb):
        row = jnp.concatenate(
            [labp[:, n45 * i:n45 * (i + 1)],
             jnp.zeros((1, 3), jnp.float32)], axis=1)
        lab_sub.append(_lane_to_sublane(row, 48))
    labc = jnp.concatenate(lab_sub, axis=0)

    zr = pos_ref[...]
    cio = _iota((zr.shape[0], _N_CLASSES), 1).astype(jnp.float32)
    tmask = (cio == labc - 1.0) & (labc > 0.0)
    pr, qr, spr_pos, spr_neg = _focal_terms(zr)
    corr_terms = (0.25 * qr * qr * spr_neg
                  - 0.75 * pr * pr * spr_pos)
    corr = jnp.sum(jnp.where(tmask, corr_terms, 0.0)).reshape(1, 1)

    return corr, npos, loc_num, sel_sum


def _body(scores_ref, pos_ref, locs_ref, boxes_ref, labels_ref, priors_ref,
          boxes_sm_ref, out_ref):
    step = pl.program_id(0)
    oio = _iota((1, 128), 1)

    z = scores_ref[0]
    p, _, sp_pos, _ = _focal_terms(z)
    bg = 0.75 * jnp.sum(p * p * sp_pos)

    @pl.when(step == 0)
    def _first():
        corr, npos, loc_num, sel_sum = _assignment(
            pos_ref, locs_ref, boxes_ref, labels_ref, priors_ref,
            boxes_sm_ref)
        out_ref[...] = (jnp.where(oio == 0, bg + corr, 0.0)
                        + jnp.where(oio == 1, npos, 0.0)
                        + jnp.where(oio == 2, loc_num, 0.0)
                        + jnp.where(oio == 3, sel_sum, 0.0))

    @pl.when(step != 0)
    def _rest():
        out_ref[...] = out_ref[...] + jnp.where(oio == 0, bg, 0.0)


def _impl(predicted_locs, predicted_scores, boxes, labels, priors,
          interpret=False):
    batch = predicted_locs.shape[0]
    n_pri = predicted_locs.shape[1]
    n_cls = predicted_scores.shape[2]

    locs_t = jnp.transpose(predicted_locs, (0, 2, 1))
    priors_t = jnp.transpose(priors, (1, 0))
    pos_rows = jnp.concatenate(
        [predicted_scores[:, s:s + _K, :] for s in _SPLITS[:-1]]
        + [jnp.zeros((batch, 3, n_cls), jnp.float32)], axis=1)
    pos2 = pos_rows.reshape(batch * 48, n_cls)
    boxes32 = boxes.reshape(batch * _N_OBJ, 4)

    parts = pl.pallas_call(
        _body,
        grid=(batch,),
        in_specs=[
            pl.BlockSpec((1, n_pri, n_cls), lambda i: (i, 0, 0)),
            pl.BlockSpec((batch * 48, n_cls), lambda i: (0, 0)),
            pl.BlockSpec((batch, 4, n_pri), lambda i: (0, 0, 0)),
            pl.BlockSpec((batch * _N_OBJ, 4), lambda i: (0, 0)),
            pl.BlockSpec((batch, _N_OBJ), lambda i: (0, 0)),
            pl.BlockSpec((4, n_pri), lambda i: (0, 0)),
            pl.BlockSpec(memory_space=pltpu.SMEM),
        ],
        out_specs=pl.BlockSpec((1, 128), lambda i: (0, 0)),
        out_shape=jax.ShapeDtypeStruct((1, 128), jnp.float32),
        interpret=interpret,
    )(predicted_scores, pos2, locs_t, boxes32, labels, priors_t, boxes32)

    focal = parts[0, 0]
    npos = jnp.maximum(parts[0, 1], 1.0)
    loc_num = parts[0, 2]
    sel_sum = jnp.maximum(parts[0, 3], 1.0)
    return focal / npos + loc_num / sel_sum


def kernel(predicted_locs, predicted_scores, boxes, labels, priors):
    return _impl(predicted_locs, predicted_scores, boxes, labels, priors)
